# baseline (speedup 1.0000x reference)
import functools

import jax
import jax.numpy as jnp
from jax import lax
from jax.experimental import pallas as pl
from jax.experimental.pallas import tpu as pltpu

_NUM_DIFF_LAYERS = 10
_C = 0.15
_BM = 512
_NBLK = 20

_A_SCALE = 1024.0
_F8 = jnp.float8_e4m3fn
_DNT = (((1,), (1,)), ((), ()))


def _quantize_kernel(ap_ref, am_ref, out_ref):
    q = pl.program_id(1)

    @pl.when(q < 2)
    def _():
        out_ref[0] = (ap_ref[...] * _A_SCALE).astype(_F8)

    @pl.when(q >= 2)
    def _():
        out_ref[0] = (am_ref[...] * _A_SCALE).astype(_F8)


def _quantize(ap, am, bm, nblk):
    n = ap.shape[0]
    qm = bm // 2
    return pl.pallas_call(
        _quantize_kernel,
        grid=(nblk, 4),
        in_specs=[
            pl.BlockSpec((qm, n), lambda i, q: (2 * i + jnp.minimum(q, 1), 0)),
            pl.BlockSpec((qm, n),
                         lambda i, q: (2 * i + jnp.where(q == 3, 1, 0), 0)),
        ],
        out_specs=pl.BlockSpec((1, qm, n), lambda i, q: (i, q, 0)),
        out_shape=jax.ShapeDtypeStruct((nblk, 2 * bm, n), _F8),
    )(ap, am)


def _diffusion_kernel(a_ref, pm0_ref, tx_ref, p_ref, m_ref,
                      s0_ref, s1_ref, *, n, d, bm, nsteps):
    s = pl.program_id(0)
    i = pl.program_id(1)

    @pl.when(jnp.logical_and(s == 0, i == 0))
    def _():
        s0_ref[...] = pm0_ref[...]

    def body(cur_ref, nxt_ref):
        a = a_ref[0]
        pmt = cur_ref[:, :n]
        y = lax.dot_general(pmt, a, _DNT,
                            preferred_element_type=jnp.float32)
        y1 = y[:, :bm]
        y2 = y[:, bm:]
        inv = 1.0 / _A_SCALE
        tx = tx_ref[:, pl.ds(i * bm, bm)]
        newp = (y1[:d, :] + y2[d:, :]) * inv + tx
        newm = (y2[:d, :] + y1[d:, :]) * inv
        p_ref[...] = newp[None]
        m_ref[...] = newm[None]
        nxt_ref[:, pl.ds(i * bm, bm)] = jnp.concatenate(
            [newp, newm], axis=0).astype(jnp.bfloat16)

    @pl.when(lax.rem(s, 2) == 0)
    def _():
        body(s0_ref, s1_ref)

    @pl.when(lax.rem(s, 2) == 1)
    def _():
        body(s1_ref, s0_ref)


def _diffusion(a8, pm0t, txt, n, bm, nblk, nsteps):
    npad = txt.shape[1]
    d = txt.shape[0]

    def out_idx(s, i):
        return (jnp.where(s == nsteps - 1, i, 0), 0, 0)

    return pl.pallas_call(
        functools.partial(_diffusion_kernel, n=n, d=d, bm=bm, nsteps=nsteps),
        grid=(nsteps, nblk),
        in_specs=[
            pl.BlockSpec((1, 2 * bm, n), lambda s, i: (i, 0, 0)),
            pl.BlockSpec((2 * d, npad), lambda s, i: (0, 0)),
            pl.BlockSpec((d, npad), lambda s, i: (0, 0)),
        ],
        out_specs=[
            pl.BlockSpec((1, d, bm), out_idx),
            pl.BlockSpec((1, d, bm), out_idx),
        ],
        out_shape=[
            jax.ShapeDtypeStruct((nblk, d, bm), jnp.float32),
            jax.ShapeDtypeStruct((nblk, d, bm), jnp.float32),
        ],
        scratch_shapes=[
            pltpu.VMEM((2 * d, npad), jnp.bfloat16),
            pltpu.VMEM((2 * d, npad), jnp.bfloat16),
        ],
    )(a8, pm0t, txt)


def kernel(nApT, nAmT, X):
    n, d = X.shape
    npad = _BM * _NBLK
    m0 = jax.random.uniform(jax.random.key(1), X.shape, dtype=jnp.float32,
                            minval=-1.0, maxval=1.0)
    txt = jnp.pad((_C * X).T, ((0, 0), (0, npad - n)))
    pm0t = jnp.pad(jnp.concatenate([X, m0], axis=1).T.astype(jnp.bfloat16),
                   ((0, 0), (0, npad - n)))
    a8 = _quantize(nApT, nAmT, _BM, _NBLK)
    pt, mt = _diffusion(a8, pm0t, txt, n, _BM, _NBLK, _NUM_DIFF_LAYERS)
    p = pt.transpose(0, 2, 1).reshape(npad, d)[:n]
    m = mt.transpose(0, 2, 1).reshape(npad, d)[:n]
    return (p, m)
